# baseline (speedup 1.0000x reference)
import jax
import jax.numpy as jnp
from jax.experimental import pallas as pl

_N_CLASSES = 1000
_B0, _B1 = 1024, 20


def _onehot_body(idx_ref, out_ref):
    j = pl.program_id(0)
    row = idx_ref[pl.ds(j, 1), :]
    iota = jax.lax.broadcasted_iota(jnp.int32, out_ref.shape, 1)
    out_ref[...] = (iota == row[:, None, :]).astype(jnp.float32)


def kernel(input):
    idx_t = jnp.transpose(input.astype(jnp.int32))
    out_t = pl.pallas_call(
        _onehot_body,
        grid=(_B1,),
        in_specs=[pl.BlockSpec((_B1, _B0), lambda j: (0, 0))],
        out_specs=pl.BlockSpec((1, _N_CLASSES, _B0), lambda j: (j, 0, 0)),
        out_shape=jax.ShapeDtypeStruct((_B1, _N_CLASSES, _B0), jnp.float32),
    )(idx_t)
    return jnp.transpose(out_t, (2, 0, 1))
